# baseline (speedup 1.0000x reference)
import functools

import jax
import jax.numpy as jnp
from jax import lax
from jax.experimental import pallas as pl
from jax.experimental.pallas import tpu as pltpu
from jax.experimental.pallas import tpu_sc as plsc

N_NODES = 10000
N_EDGES = 320000
D_FEAT = 128
HIDDEN = 128
NUM_CLASSES = 64
NUM_EDGE_FEATURES = 4
NUM_BINS = 16
NUM_COLS = NUM_EDGE_FEATURES * NUM_BINS

L = 16
NC = 2
NS = 16
NW = NC * NS
CH = 8000
NCH = N_EDGES // CH
IT_PER_CH = CH // L
UNROLL = 10
NPR = 640
PCH = 4000
E_PER_TILE = N_EDGES // NS


def _sc_counts(s, f_flat):
    mesh = plsc.VectorSubcoreMesh(
        core_axis_name="c", subcore_axis_name="s", num_cores=NC,
        num_subcores=NS)

    @functools.partial(
        pl.kernel,
        mesh=mesh,
        compiler_params=pltpu.CompilerParams(needs_layout_passes=False),
        out_type=jax.ShapeDtypeStruct((NW * 2 * N_NODES,), jnp.float32),
        scratch_types=[
            pltpu.VMEM((N_NODES,), jnp.int32),
            pltpu.VMEM((2 * N_NODES,), jnp.float32),
            pltpu.VMEM((CH,), jnp.int32),
            pltpu.VMEM((CH,), jnp.int32),
            pltpu.VMEM((CH,), jnp.int32),
            pltpu.VMEM((CH,), jnp.int32),
            pltpu.VMEM((NPR * L,), jnp.int32),
            pltpu.SemaphoreType.DMA,
            pltpu.SemaphoreType.DMA,
        ],
    )
    def body(s_hbm, f_hbm, out_hbm, s0_v, hist_v, sbuf0, sbuf1, fbuf0, fbuf1,
             pres_v, sem0, sem1):
        cid = lax.axis_index("c")
        sid = lax.axis_index("s")
        wid = sid * NC + cid
        i_grp = wid // (NW // NUM_EDGE_FEATURES)
        b_lo = (2 * wid) % NUM_BINS
        b_hi = b_lo + 1
        f_base = i_grp * N_EDGES

        sbuf = (sbuf0, sbuf1)
        fbuf = (fbuf0, fbuf1)
        sem = (sem0, sem1)

        def issue(ci, b):
            off = ci * CH
            pltpu.async_copy(s_hbm.at[pl.ds(off, CH)], sbuf[b], sem[b])
            pltpu.async_copy(f_hbm.at[pl.ds(f_base + off, CH)], fbuf[b],
                             sem[b])

        def wait(b):
            pltpu.make_async_copy(s_hbm.at[pl.ds(0, CH)], sbuf[b],
                                  sem[b]).wait()
            pltpu.make_async_copy(f_hbm.at[pl.ds(0, CH)], fbuf[b],
                                  sem[b]).wait()

        def process(b):
            def it(j, _):
                base0 = j * (UNROLL * L)
                svs = [sbuf[b][pl.ds(base0 + u * L, L)]
                       for u in range(UNROLL)]
                fvs = [fbuf[b][pl.ds(base0 + u * L, L)]
                       for u in range(UNROLL)]
                ts = [plsc.load_gather(s0_v, [sv]) for sv in svs]
                m_his = [fv == b_hi for fv in fvs]
                ms = [(fv == b_lo) | mh for fv, mh in zip(fvs, m_his)]
                idxs = [t + jnp.where(mh, N_NODES, 0)
                        for t, mh in zip(ts, m_his)]
                scans = [plsc.scan_count(ix, m) for ix, m in zip(idxs, ms)]
                for ix, (cnt, last) in zip(idxs, scans):
                    plsc.addupdate_scatter(
                        hist_v, [ix], cnt.astype(jnp.float32), mask=last)
                return 0

            lax.fori_loop(0, IT_PER_CH // UNROLL, it, 0)

        zeros = jnp.zeros((L,), jnp.float32)
        izero = jnp.zeros((L,), jnp.int32)
        ione = jnp.ones((L,), jnp.int32)
        lane = lax.iota(jnp.int32, L)

        def zero_body(j, _):
            hist_v[pl.ds(j * L, L)] = zeros
            return 0

        lax.fori_loop(0, 2 * N_NODES // L, zero_body, 0)

        pltpu.sync_copy(s_hbm.at[pl.ds(0, N_NODES)], s0_v)

        def zero_pres(j, _):
            pres_v[pl.ds(j * L, L)] = izero
            return 0

        lax.fori_loop(0, NPR, zero_pres, 0)

        def pres_issue(ci, b):
            pltpu.async_copy(s_hbm.at[pl.ds(ci * CH, CH)], sbuf[b], sem[b])

        def pres_wait(b):
            pltpu.make_async_copy(s_hbm.at[pl.ds(0, CH)], sbuf[b],
                                  sem[b]).wait()

        def pres_scan(b):
            def it(j, _):
                base0 = j * (UNROLL * L)
                svs = [sbuf[b][pl.ds(base0 + u * L, L)]
                       for u in range(UNROLL)]
                scans = [plsc.scan_count(sv) for sv in svs]
                for sv, (cnt, last) in zip(svs, scans):
                    plsc.store_scatter(pres_v, [sv], ione, mask=last)
                return 0

            lax.fori_loop(0, IT_PER_CH // UNROLL, it, 0)

        pres_issue(0, 0)
        pres_issue(1, 1)

        def pres_outer(k, _):
            c0 = 2 * k
            pres_wait(0)
            pres_scan(0)

            @pl.when(c0 + 2 < NCH)
            def _():
                pres_issue(c0 + 2, 0)

            pres_wait(1)
            pres_scan(1)

            @pl.when(c0 + 3 < NCH)
            def _():
                pres_issue(c0 + 3, 1)

            return 0

        lax.fori_loop(0, NCH // 2, pres_outer, 0)

        def rank_body(j, carry):
            p = pres_v[pl.ds(j * L, L)]
            cs = plsc.cumsum(p)
            pres_v[pl.ds(j * L, L)] = cs - p + carry
            return carry + lax.reduce_sum(p, axes=(0,))

        lax.fori_loop(0, N_NODES // L, rank_body, jnp.int32(0))

        def m0_body(j, _):
            v = s0_v[pl.ds(j * L, L)]
            r = plsc.load_gather(pres_v, [v])
            s0_v[pl.ds(j * L, L)] = r
            return 0

        lax.fori_loop(0, N_NODES // L, m0_body, 0)

        issue(0, 0)
        issue(1, 1)

        def outer(k, _):
            c0 = 2 * k
            wait(0)
            process(0)

            @pl.when(c0 + 2 < NCH)
            def _():
                issue(c0 + 2, 0)

            wait(1)
            process(1)

            @pl.when(c0 + 3 < NCH)
            def _():
                issue(c0 + 3, 1)

            return 0

        lax.fori_loop(0, NCH // 2, outer, 0)

        pltpu.sync_copy(hist_v, out_hbm.at[pl.ds(wid * 2 * N_NODES,
                                                 2 * N_NODES)])

    return body(s, f_flat)


def _tc_dense(x, W1, b1, aggT, W2a, W2b, b2):
    def body(x_ref, w1_ref, b1_ref, aggt_ref, w2a_ref, w2b_ref, b2_ref,
             o_ref):
        h = jnp.maximum(
            jnp.dot(x_ref[...], w1_ref[...],
                    preferred_element_type=jnp.float32,
                    precision=lax.Precision.HIGHEST) + b1_ref[...], 0.0)
        agg_term = lax.dot_general(
            aggt_ref[...], w2b_ref[...],
            dimension_numbers=(((0,), (0,)), ((), ())),
            preferred_element_type=jnp.float32,
            precision=lax.Precision.HIGHEST)
        o_ref[...] = (
            jnp.dot(h, w2a_ref[...], preferred_element_type=jnp.float32,
                    precision=lax.Precision.HIGHEST)
            + agg_term + b2_ref[...])

    return pl.pallas_call(
        body,
        out_shape=jax.ShapeDtypeStruct((N_NODES, NUM_CLASSES), jnp.float32),
    )(x, W1, b1, aggT, W2a, W2b, b2)


def kernel(x, edge_index, edge_features, W1, b1, W2, b2):
    s = edge_index[:, 0]
    f_flat = edge_features.T.reshape(-1)
    counts = _sc_counts(s, f_flat)
    aggT = counts.reshape(NUM_COLS, N_NODES)
    return _tc_dense(x, W1, b1.reshape(1, HIDDEN), aggT,
                     W2[:HIDDEN], W2[HIDDEN:], b2.reshape(1, NUM_CLASSES))


# speedup vs baseline: 1.3864x; 1.3864x over previous
import functools

import jax
import jax.numpy as jnp
from jax import lax
from jax.experimental import pallas as pl
from jax.experimental.pallas import tpu as pltpu
from jax.experimental.pallas import tpu_sc as plsc

N_NODES = 10000
N_EDGES = 320000
D_FEAT = 128
HIDDEN = 128
NUM_CLASSES = 64
NUM_EDGE_FEATURES = 4
NUM_BINS = 16
NUM_COLS = NUM_EDGE_FEATURES * NUM_BINS

L = 16
NC = 2
NS = 16
NW = NC * NS
CH = 8000
NCH = N_EDGES // CH
IT_PER_CH = CH // L
UNROLL = 10
NPR = 640


def _sc_counts(s, f_flat):
    mesh = plsc.VectorSubcoreMesh(
        core_axis_name="c", subcore_axis_name="s", num_cores=NC,
        num_subcores=NS)

    @functools.partial(
        pl.kernel,
        mesh=mesh,
        compiler_params=pltpu.CompilerParams(needs_layout_passes=False),
        out_type=jax.ShapeDtypeStruct((NW * 2 * N_NODES,), jnp.float32),
        scratch_types=[
            pltpu.VMEM((N_NODES,), jnp.int32),
            pltpu.VMEM((2 * N_NODES,), jnp.float32),
            pltpu.VMEM((2 * N_NODES,), jnp.float32),
            pltpu.VMEM((CH,), jnp.int32),
            pltpu.VMEM((CH,), jnp.int32),
            pltpu.VMEM((CH,), jnp.int32),
            pltpu.VMEM((CH,), jnp.int32),
            pltpu.VMEM((NPR * L,), jnp.int32),
            pltpu.SemaphoreType.DMA,
            pltpu.SemaphoreType.DMA,
        ],
    )
    def body(s_hbm, f_hbm, out_hbm, s0_v, hist_v, hist2_v, sbuf0, sbuf1,
             fbuf0, fbuf1, pres_v, sem0, sem1):
        cid = lax.axis_index("c")
        sid = lax.axis_index("s")
        wid = sid * NC + cid
        i_grp = wid // (NW // NUM_EDGE_FEATURES)
        b_lo = (2 * wid) % NUM_BINS
        b_hi = b_lo + 1
        f_base = i_grp * N_EDGES

        sbuf = (sbuf0, sbuf1)
        fbuf = (fbuf0, fbuf1)
        sem = (sem0, sem1)

        def issue(ci, b):
            off = ci * CH
            pltpu.async_copy(s_hbm.at[pl.ds(off, CH)], sbuf[b], sem[b])
            pltpu.async_copy(f_hbm.at[pl.ds(f_base + off, CH)], fbuf[b],
                             sem[b])

        def wait(b):
            pltpu.make_async_copy(s_hbm.at[pl.ds(0, CH)], sbuf[b],
                                  sem[b]).wait()
            pltpu.make_async_copy(f_hbm.at[pl.ds(0, CH)], fbuf[b],
                                  sem[b]).wait()

        def process(b):
            def it(j, _):
                base0 = j * (UNROLL * L)
                svs = [sbuf[b][pl.ds(base0 + u * L, L)]
                       for u in range(UNROLL)]
                fvs = [fbuf[b][pl.ds(base0 + u * L, L)]
                       for u in range(UNROLL)]
                ts = [plsc.load_gather(s0_v, [sv]) for sv in svs]
                m_his = [fv == b_hi for fv in fvs]
                ms = [(fv == b_lo) | mh for fv, mh in zip(fvs, m_his)]
                idxs = [t + jnp.where(mh, N_NODES, 0)
                        for t, mh in zip(ts, m_his)]
                pscans = [plsc.scan_count(sv) for sv in svs]
                scans = [plsc.scan_count(ix, m) for ix, m in zip(idxs, ms)]
                for sv, (_, plast) in zip(svs, pscans):
                    plsc.store_scatter(pres_v, [sv], ione, mask=plast)
                for ix, (cnt, last) in zip(idxs, scans):
                    plsc.addupdate_scatter(
                        hist_v, [ix], cnt.astype(jnp.float32), mask=last)
                return 0

            lax.fori_loop(0, IT_PER_CH // UNROLL, it, 0)

        zeros = jnp.zeros((L,), jnp.float32)
        izero = jnp.zeros((L,), jnp.int32)
        ione = jnp.ones((L,), jnp.int32)

        issue(0, 0)
        issue(1, 1)

        def zero_body(j, _):
            hist_v[pl.ds(j * L, L)] = zeros
            hist2_v[pl.ds(j * L, L)] = zeros
            return 0

        lax.fori_loop(0, 2 * N_NODES // L, zero_body, 0)

        def zero_pres(j, _):
            pres_v[pl.ds(j * L, L)] = izero
            return 0

        lax.fori_loop(0, NPR, zero_pres, 0)

        pltpu.sync_copy(s_hbm.at[pl.ds(0, N_NODES)], s0_v)

        def outer(k, _):
            c0 = 2 * k
            wait(0)
            process(0)

            @pl.when(c0 + 2 < NCH)
            def _():
                issue(c0 + 2, 0)

            wait(1)
            process(1)

            @pl.when(c0 + 3 < NCH)
            def _():
                issue(c0 + 3, 1)

            return 0

        lax.fori_loop(0, NCH // 2, outer, 0)

        def rank_body(j, carry):
            p = pres_v[pl.ds(j * L, L)]
            cs = plsc.cumsum(p)
            r = cs - p + carry
            pm = p == 1
            h0 = hist_v[pl.ds(j * L, L)]
            h1 = hist_v[pl.ds(N_NODES + j * L, L)]
            plsc.store_scatter(hist2_v, [r], h0, mask=pm)
            plsc.store_scatter(hist2_v, [r + N_NODES], h1, mask=pm)
            return carry + lax.reduce_sum(p, axes=(0,))

        lax.fori_loop(0, N_NODES // L, rank_body, jnp.int32(0))

        pltpu.sync_copy(hist2_v, out_hbm.at[pl.ds(wid * 2 * N_NODES,
                                                  2 * N_NODES)])

    return body(s, f_flat)


def _tc_dense(x, W1, b1, aggT, W2a, W2b, b2):
    def body(x_ref, w1_ref, b1_ref, aggt_ref, w2a_ref, w2b_ref, b2_ref,
             o_ref):
        h = jnp.maximum(
            jnp.dot(x_ref[...], w1_ref[...],
                    preferred_element_type=jnp.float32) + b1_ref[...], 0.0)
        agg_term = lax.dot_general(
            aggt_ref[...], w2b_ref[...],
            dimension_numbers=(((0,), (0,)), ((), ())),
            preferred_element_type=jnp.float32,
            precision=lax.Precision.HIGHEST)
        o_ref[...] = (
            jnp.dot(h, w2a_ref[...], preferred_element_type=jnp.float32)
            + agg_term + b2_ref[...])

    return pl.pallas_call(
        body,
        out_shape=jax.ShapeDtypeStruct((N_NODES, NUM_CLASSES), jnp.float32),
    )(x, W1, b1, aggT, W2a, W2b, b2)


def kernel(x, edge_index, edge_features, W1, b1, W2, b2):
    s = edge_index[:, 0]
    f_flat = edge_features.T.reshape(-1)
    counts = _sc_counts(s, f_flat)
    aggT = counts.reshape(NUM_COLS, N_NODES)
    return _tc_dense(x, W1, b1.reshape(1, HIDDEN), aggT,
                     W2[:HIDDEN], W2[HIDDEN:], b2.reshape(1, NUM_CLASSES))


# speedup vs baseline: 1.4362x; 1.0359x over previous
import functools

import jax
import jax.numpy as jnp
from jax import lax
from jax.experimental import pallas as pl
from jax.experimental.pallas import tpu as pltpu
from jax.experimental.pallas import tpu_sc as plsc

N_NODES = 10000
N_EDGES = 320000
D_FEAT = 128
HIDDEN = 128
NUM_CLASSES = 64
NUM_EDGE_FEATURES = 4
NUM_BINS = 16
NUM_COLS = NUM_EDGE_FEATURES * NUM_BINS

L = 16
NC = 2
NS = 16
NW = NC * NS
CH = 16000
NCH = N_EDGES // CH
IT_PER_CH = CH // L
UNROLL = 10
NPR = 640


def _sc_counts(s, f_flat):
    mesh = plsc.VectorSubcoreMesh(
        core_axis_name="c", subcore_axis_name="s", num_cores=NC,
        num_subcores=NS)

    @functools.partial(
        pl.kernel,
        mesh=mesh,
        compiler_params=pltpu.CompilerParams(needs_layout_passes=False),
        out_type=jax.ShapeDtypeStruct((NW * 2 * N_NODES,), jnp.float32),
        scratch_types=[
            pltpu.VMEM((N_NODES,), jnp.int32),
            pltpu.VMEM((2 * N_NODES,), jnp.float32),
            pltpu.VMEM((2 * N_NODES,), jnp.float32),
            pltpu.VMEM((CH,), jnp.int32),
            pltpu.VMEM((CH,), jnp.int32),
            pltpu.VMEM((CH,), jnp.int32),
            pltpu.VMEM((CH,), jnp.int32),
            pltpu.VMEM((NPR * L,), jnp.int32),
            pltpu.SemaphoreType.DMA,
            pltpu.SemaphoreType.DMA,
        ],
    )
    def body(s_hbm, f_hbm, out_hbm, s0_v, hist_v, hist2_v, sbuf0, sbuf1,
             fbuf0, fbuf1, pres_v, sem0, sem1):
        cid = lax.axis_index("c")
        sid = lax.axis_index("s")
        wid = sid * NC + cid
        i_grp = wid // (NW // NUM_EDGE_FEATURES)
        b_lo = (2 * wid) % NUM_BINS
        b_hi = b_lo + 1
        f_base = i_grp * N_EDGES

        sbuf = (sbuf0, sbuf1)
        fbuf = (fbuf0, fbuf1)
        sem = (sem0, sem1)

        def issue(ci, b):
            off = ci * CH
            pltpu.async_copy(s_hbm.at[pl.ds(off, CH)], sbuf[b], sem[b])
            pltpu.async_copy(f_hbm.at[pl.ds(f_base + off, CH)], fbuf[b],
                             sem[b])

        def wait(b):
            pltpu.make_async_copy(s_hbm.at[pl.ds(0, CH)], sbuf[b],
                                  sem[b]).wait()
            pltpu.make_async_copy(f_hbm.at[pl.ds(0, CH)], fbuf[b],
                                  sem[b]).wait()

        def process(b):
            def it(j, _):
                base0 = j * (UNROLL * L)
                svs = [sbuf[b][pl.ds(base0 + u * L, L)]
                       for u in range(UNROLL)]
                fvs = [fbuf[b][pl.ds(base0 + u * L, L)]
                       for u in range(UNROLL)]
                ts = [plsc.load_gather(s0_v, [sv]) for sv in svs]
                m_his = [fv == b_hi for fv in fvs]
                ms = [(fv == b_lo) | mh for fv, mh in zip(fvs, m_his)]
                idxs = [t + jnp.where(mh, N_NODES, 0)
                        for t, mh in zip(ts, m_his)]
                scans = [plsc.scan_count(ix, m) for ix, m in zip(idxs, ms)]
                for sv in svs:
                    plsc.store_scatter(pres_v, [sv], ione)
                for ix, (cnt, last) in zip(idxs, scans):
                    plsc.addupdate_scatter(
                        hist_v, [ix], cnt.astype(jnp.float32), mask=last)
                return 0

            lax.fori_loop(0, IT_PER_CH // UNROLL, it, 0)

        zeros = jnp.zeros((L,), jnp.float32)
        izero = jnp.zeros((L,), jnp.int32)
        ione = jnp.ones((L,), jnp.int32)

        issue(0, 0)
        issue(1, 1)

        def zero_body(j, _):
            hist_v[pl.ds(j * L, L)] = zeros
            hist2_v[pl.ds(j * L, L)] = zeros
            return 0

        lax.fori_loop(0, 2 * N_NODES // L, zero_body, 0)

        def zero_pres(j, _):
            pres_v[pl.ds(j * L, L)] = izero
            return 0

        lax.fori_loop(0, NPR, zero_pres, 0)

        pltpu.sync_copy(s_hbm.at[pl.ds(0, N_NODES)], s0_v)

        def outer(k, _):
            c0 = 2 * k
            wait(0)
            process(0)

            @pl.when(c0 + 2 < NCH)
            def _():
                issue(c0 + 2, 0)

            wait(1)
            process(1)

            @pl.when(c0 + 3 < NCH)
            def _():
                issue(c0 + 3, 1)

            return 0

        lax.fori_loop(0, NCH // 2, outer, 0)

        def rank_body(j, carry):
            p = pres_v[pl.ds(j * L, L)]
            cs = plsc.cumsum(p)
            r = cs - p + carry
            pm = p == 1
            h0 = hist_v[pl.ds(j * L, L)]
            h1 = hist_v[pl.ds(N_NODES + j * L, L)]
            plsc.store_scatter(hist2_v, [r], h0, mask=pm)
            plsc.store_scatter(hist2_v, [r + N_NODES], h1, mask=pm)
            return carry + lax.reduce_sum(p, axes=(0,))

        lax.fori_loop(0, N_NODES // L, rank_body, jnp.int32(0))

        pltpu.sync_copy(hist2_v, out_hbm.at[pl.ds(wid * 2 * N_NODES,
                                                  2 * N_NODES)])

    return body(s, f_flat)


def _tc_dense(x, W1, b1, aggT, W2a, W2b, b2):
    def body(x_ref, w1_ref, b1_ref, aggt_ref, w2a_ref, w2b_ref, b2_ref,
             o_ref):
        h = jnp.maximum(
            jnp.dot(x_ref[...], w1_ref[...],
                    preferred_element_type=jnp.float32) + b1_ref[...], 0.0)
        agg_term = lax.dot_general(
            aggt_ref[...], w2b_ref[...],
            dimension_numbers=(((0,), (0,)), ((), ())),
            preferred_element_type=jnp.float32)
        o_ref[...] = (
            jnp.dot(h, w2a_ref[...], preferred_element_type=jnp.float32)
            + agg_term + b2_ref[...])

    return pl.pallas_call(
        body,
        out_shape=jax.ShapeDtypeStruct((N_NODES, NUM_CLASSES), jnp.float32),
    )(x, W1, b1, aggT, W2a, W2b, b2)


def kernel(x, edge_index, edge_features, W1, b1, W2, b2):
    s = edge_index[:, 0]
    f_flat = edge_features.T.reshape(-1)
    counts = _sc_counts(s, f_flat)
    aggT = counts.reshape(NUM_COLS, N_NODES)
    return _tc_dense(x, W1, b1.reshape(1, HIDDEN), aggT,
                     W2[:HIDDEN], W2[HIDDEN:], b2.reshape(1, NUM_CLASSES))
